# Initial kernel scaffold; baseline (speedup 1.0000x reference)
import jax
import jax.numpy as jnp
from jax.experimental import pallas as pl


def kernel(x, edge_index, W1, b1, W2, b2, W3, b3):
    raise NotImplementedError("write your pallas kernel here")


# speedup vs baseline: 21.5546x; 21.5546x over previous
import functools

import jax
import jax.numpy as jnp
from jax import lax
from jax.experimental import pallas as pl
from jax.experimental.pallas import tpu as pltpu
from jax.experimental.pallas import tpu_sc as plsc

NN = 100000
EE = 1600000
NC = 2
NS = 16


DEG_PER_TILE = 6256
DEG_PAD = NS * DEG_PER_TILE
DEG_EPT = EE // (NC * NS)
DEG_BLK = 10000


def _sc_degree(dst):
    mesh = plsc.VectorSubcoreMesh(core_axis_name="c", subcore_axis_name="s")

    @functools.partial(
        pl.kernel,
        out_type=jax.ShapeDtypeStruct((NC * DEG_PAD,), jnp.float32),
        mesh=mesh,
        scratch_types=[
            pltpu.VMEM((DEG_BLK,), jnp.int32),
            pltpu.VMEM((DEG_BLK,), jnp.float32),
            pltpu.VMEM((DEG_PER_TILE,), jnp.float32),
            pltpu.VMEM_SHARED((DEG_PAD,), jnp.float32),
        ],
    )
    def k(dst_hbm, out_hbm, dblk, ones, zbuf, deg_sh):
        cid = lax.axis_index("c")
        sid = lax.axis_index("s")

        @pl.loop(0, DEG_BLK, step=16)
        def _(i):
            ones[pl.ds(i, 16)] = jnp.full((16,), 1.0, jnp.float32)

        @pl.loop(0, DEG_PER_TILE, step=16)
        def _(i):
            zbuf[pl.ds(i, 16)] = jnp.zeros((16,), jnp.float32)

        pltpu.sync_copy(zbuf, deg_sh.at[pl.ds(sid * DEG_PER_TILE, DEG_PER_TILE)])
        plsc.subcore_barrier()

        base = (cid * NS + sid) * DEG_EPT

        @pl.loop(0, DEG_EPT, step=DEG_BLK)
        def _(e0):
            pltpu.sync_copy(dst_hbm.at[pl.ds(base + e0, DEG_BLK)], dblk)
            pltpu.sync_copy(ones, deg_sh.at[dblk], add=True)

        plsc.subcore_barrier()
        pltpu.sync_copy(deg_sh.at[pl.ds(sid * DEG_PER_TILE, DEG_PER_TILE)], zbuf)
        pltpu.sync_copy(
            zbuf,
            out_hbm.at[pl.ds(cid * DEG_PAD + sid * DEG_PER_TILE, DEG_PER_TILE)],
        )

    return k(dst)


EPT = EE // NS
KBLK = 2000
G = 256
GSH = 8
NGMAX = (KBLK + G - 1) // G
CFLAT = NGMAX * G + 16


def _make_sc_agg(D, C, n_chunks):
    PAD = 16 + ((16 - C % 16) % 16)
    C_pad = C + PAD
    RPT = C_pad // NS
    NOF, NOR = RPT // G, RPT % G
    DV = D // 16
    mesh = plsc.VectorSubcoreMesh(core_axis_name="c", subcore_axis_name="s")

    @functools.partial(
        pl.kernel,
        out_type=jax.ShapeDtypeStruct((n_chunks * C_pad, D), jnp.float32),
        mesh=mesh,
        scratch_types=[
            pltpu.VMEM((KBLK,), jnp.int32),
            pltpu.VMEM((KBLK,), jnp.int32),
            pltpu.VMEM((CFLAT,), jnp.int32),
            pltpu.VMEM((CFLAT,), jnp.int32),
            pltpu.VMEM((NGMAX, G), jnp.int32),
            pltpu.VMEM((G, D), jnp.float32),
            pltpu.VMEM_SHARED((C_pad, D), jnp.float32),
        ],
        compiler_params=pltpu.CompilerParams(use_tc_tiling_on_sc=False,
                                             needs_layout_passes=False),
    )
    def k(h_hbm, src_hbm, dst_hbm, agg_hbm,
          dblk, sblk, csrc, cdst, cdst2d, rows, chunk_sh):
        cid = lax.axis_index("c")
        sid = lax.axis_index("s")
        ebase = sid * EPT
        dump_row = C + sid
        pad_src = lax.iota(jnp.int32, 16) + sid * 16

        for p in range(n_chunks // NC):
            ch = cid + NC * p
            lo = ch * C
            row0 = sid * RPT

            @pl.loop(0, G)
            def _(r):
                for j in range(DV):
                    rows[r, pl.ds(j * 16, 16)] = jnp.zeros((16,), jnp.float32)

            @pl.loop(0, NOF)
            def _(z):
                pltpu.sync_copy(rows, chunk_sh.at[pl.ds(row0 + z * G, G)])
            if NOR:
                pltpu.sync_copy(rows.at[pl.ds(0, NOR)],
                                chunk_sh.at[pl.ds(row0 + NOF * G, NOR)])
            plsc.subcore_barrier()

            def block(b, carry):
                e0 = ebase + b * KBLK
                pltpu.sync_copy(src_hbm.at[pl.ds(e0, KBLK)], sblk)
                pltpu.sync_copy(dst_hbm.at[pl.ds(e0, KBLK)], dblk)

                def comp(i, m):
                    dv = dblk[pl.ds(i * 16, 16)]
                    sv = sblk[pl.ds(i * 16, 16)]
                    msk = (dv >= lo) & (dv < lo + C)
                    plsc.store_compressed(csrc.at[pl.ds(m, 16)], sv, mask=msk)
                    plsc.store_compressed(cdst.at[pl.ds(m, 16)], dv - lo, mask=msk)
                    return m + jnp.sum(msk.astype(jnp.int32))

                m = lax.fori_loop(0, KBLK // 16, comp, jnp.int32(0))
                ng = (m + (G - 1)) >> GSH
                npad = ((ng << GSH) - m + 15) >> 4
                pad_dst = jnp.full((16,), 0, jnp.int32) + dump_row

                def padb(i, _):
                    csrc[pl.ds(m + i * 16, 16)] = pad_src
                    cdst[pl.ds(m + i * 16, 16)] = pad_dst
                    return 0

                lax.fori_loop(0, npad, padb, 0)

                def cpb(i, _):
                    g = i >> (GSH - 4)
                    j = i & (G // 16 - 1)
                    cdst2d[g, pl.ds(j * 16, 16)] = cdst[pl.ds(i * 16, 16)]
                    return 0

                lax.fori_loop(0, ng << (GSH - 4), cpb, 0)

                def grp(g, _):
                    pltpu.sync_copy(h_hbm.at[csrc.at[pl.ds(g * G, G)]], rows)
                    pltpu.sync_copy(rows, chunk_sh.at[cdst2d.at[g]], add=True)
                    return 0

                lax.fori_loop(0, ng, grp, 0)
                return carry

            lax.fori_loop(0, EPT // KBLK, block, 0)
            plsc.subcore_barrier()

            @pl.loop(0, NOF)
            def _(z):
                pltpu.sync_copy(chunk_sh.at[pl.ds(row0 + z * G, G)], rows)
                pltpu.sync_copy(
                    rows, agg_hbm.at[pl.ds(ch * C_pad + row0 + z * G, G)])
            if NOR:
                pltpu.sync_copy(chunk_sh.at[pl.ds(row0 + NOF * G, NOR)],
                                rows.at[pl.ds(0, NOR)])
                pltpu.sync_copy(
                    rows.at[pl.ds(0, NOR)],
                    agg_hbm.at[pl.ds(ch * C_pad + row0 + NOF * G, NOR)])
            plsc.subcore_barrier()

    return k


def _sc_agg(hprime, src, dst, D, C, n_chunks):
    PAD = 16 + ((16 - C % 16) % 16)
    C_pad = C + PAD
    out = _make_sc_agg(D, C, n_chunks)(hprime, src, dst)
    return out.reshape(n_chunks, C_pad, D)[:, :C, :].reshape(n_chunks * C, D)


RB = 2000


def _tc_mm(x, W, dout):
    din = x.shape[1]

    def body(x_ref, w_ref, o_ref):
        o_ref[...] = jnp.dot(x_ref[...], w_ref[...],
                             preferred_element_type=jnp.float32)

    return pl.pallas_call(
        body,
        grid=(NN // RB,),
        in_specs=[pl.BlockSpec((RB, din), lambda i: (i, 0)),
                  pl.BlockSpec((din, dout), lambda i: (0, 0))],
        out_specs=pl.BlockSpec((RB, dout), lambda i: (i, 0)),
        out_shape=jax.ShapeDtypeStruct((NN, dout), jnp.float32),
    )(x, W)


def _tc_scale(h, dinv2d):
    D = h.shape[1]

    def body(h_ref, dv_ref, o_ref):
        o_ref[...] = h_ref[...] * dv_ref[...]

    return pl.pallas_call(
        body,
        grid=(NN // RB,),
        in_specs=[pl.BlockSpec((RB, D), lambda i: (i, 0)),
                  pl.BlockSpec((RB, 1), lambda i: (i, 0))],
        out_specs=pl.BlockSpec((RB, D), lambda i: (i, 0)),
        out_shape=jax.ShapeDtypeStruct((NN, D), jnp.float32),
    )(h, dinv2d)


def _tc_junction(agg, hp, dinv2d, b, Wn):
    D = agg.shape[1]
    Dn = Wn.shape[1]

    def body(a_ref, h_ref, dv_ref, b_ref, w_ref, o_ref):
        dv = dv_ref[...]
        t = jax.nn.relu(dv * (a_ref[...] + h_ref[...]) + b_ref[...])
        o_ref[...] = jnp.dot(t, w_ref[...],
                             preferred_element_type=jnp.float32) * dv

    return pl.pallas_call(
        body,
        grid=(NN // RB,),
        in_specs=[pl.BlockSpec((RB, D), lambda i: (i, 0)),
                  pl.BlockSpec((RB, D), lambda i: (i, 0)),
                  pl.BlockSpec((RB, 1), lambda i: (i, 0)),
                  pl.BlockSpec((1, D), lambda i: (0, 0)),
                  pl.BlockSpec((D, Dn), lambda i: (0, 0))],
        out_specs=pl.BlockSpec((RB, Dn), lambda i: (i, 0)),
        out_shape=jax.ShapeDtypeStruct((NN, Dn), jnp.float32),
    )(agg, hp, dinv2d, b.reshape(1, D), Wn)


def _tc_post(agg, hp, dinv2d, b):
    D = agg.shape[1]

    def body(a_ref, h_ref, dv_ref, b_ref, o_ref):
        o_ref[...] = dv_ref[...] * (a_ref[...] + h_ref[...]) + b_ref[...]

    return pl.pallas_call(
        body,
        grid=(NN // RB,),
        in_specs=[pl.BlockSpec((RB, D), lambda i: (i, 0)),
                  pl.BlockSpec((RB, D), lambda i: (i, 0)),
                  pl.BlockSpec((RB, 1), lambda i: (i, 0)),
                  pl.BlockSpec((1, D), lambda i: (0, 0))],
        out_specs=pl.BlockSpec((RB, D), lambda i: (i, 0)),
        out_shape=jax.ShapeDtypeStruct((NN, D), jnp.float32),
    )(agg, hp, dinv2d, b.reshape(1, D))


def kernel(x, edge_index, W1, b1, W2, b2, W3, b3):
    src = edge_index[0]
    dst = edge_index[1]

    degp = _sc_degree(dst)
    deg = degp[:NN] + degp[DEG_PAD:DEG_PAD + NN] + 1.0
    dinv2d = lax.rsqrt(deg).reshape(NN, 1)

    h1p = _tc_scale(_tc_mm(x, W1, 16), dinv2d)
    agg1 = _sc_agg(h1p, src, dst, 16, 50000, 2)
    h2p = _tc_junction(agg1, h1p, dinv2d, b1, W2)
    agg2 = _sc_agg(h2p, src, dst, 32, 50000, 2)
    h3p = _tc_junction(agg2, h2p, dinv2d, b2, W3)
    agg3 = _sc_agg(h3p, src, dst, 64, 25000, 4)
    return _tc_post(agg3, h3p, dinv2d, b3)


# speedup vs baseline: 24.8321x; 1.1521x over previous
import functools

import jax
import jax.numpy as jnp
from jax import lax
from jax.experimental import pallas as pl
from jax.experimental.pallas import tpu as pltpu
from jax.experimental.pallas import tpu_sc as plsc

NN = 100000
EE = 1600000
NC = 2
NS = 16


DEG_PER_TILE = 6256
DEG_PAD = NS * DEG_PER_TILE
DEG_EPT = EE // (NC * NS)
DEG_BLK = 10000


def _sc_degree(dst):
    mesh = plsc.VectorSubcoreMesh(core_axis_name="c", subcore_axis_name="s")

    @functools.partial(
        pl.kernel,
        out_type=jax.ShapeDtypeStruct((NC * DEG_PAD,), jnp.float32),
        mesh=mesh,
        scratch_types=[
            pltpu.VMEM((DEG_BLK,), jnp.int32),
            pltpu.VMEM((DEG_BLK,), jnp.float32),
            pltpu.VMEM((DEG_PER_TILE,), jnp.float32),
            pltpu.VMEM_SHARED((DEG_PAD,), jnp.float32),
        ],
    )
    def k(dst_hbm, out_hbm, dblk, ones, zbuf, deg_sh):
        cid = lax.axis_index("c")
        sid = lax.axis_index("s")

        @pl.loop(0, DEG_BLK, step=16)
        def _(i):
            ones[pl.ds(i, 16)] = jnp.full((16,), 1.0, jnp.float32)

        @pl.loop(0, DEG_PER_TILE, step=16)
        def _(i):
            zbuf[pl.ds(i, 16)] = jnp.zeros((16,), jnp.float32)

        pltpu.sync_copy(zbuf, deg_sh.at[pl.ds(sid * DEG_PER_TILE, DEG_PER_TILE)])
        plsc.subcore_barrier()

        base = (cid * NS + sid) * DEG_EPT

        @pl.loop(0, DEG_EPT, step=DEG_BLK)
        def _(e0):
            pltpu.sync_copy(dst_hbm.at[pl.ds(base + e0, DEG_BLK)], dblk)
            pltpu.sync_copy(ones, deg_sh.at[dblk], add=True)

        plsc.subcore_barrier()
        pltpu.sync_copy(deg_sh.at[pl.ds(sid * DEG_PER_TILE, DEG_PER_TILE)], zbuf)
        pltpu.sync_copy(
            zbuf,
            out_hbm.at[pl.ds(cid * DEG_PAD + sid * DEG_PER_TILE, DEG_PER_TILE)],
        )

    return k(dst)


EPT = EE // NS
KBLK = 2000


def _make_sc_agg(D, C, n_chunks):
    PAD = 16 + ((16 - C % 16) % 16)
    C_pad = C + PAD
    RPT = C_pad // NS
    G = 128 if D == 64 else 256
    GSH = G.bit_length() - 1
    NGMAX = (KBLK + G - 1) // G
    CFLAT = NGMAX * G + 16
    NOF, NOR = RPT // G, RPT % G
    DV = D // 16
    mesh = plsc.VectorSubcoreMesh(core_axis_name="c", subcore_axis_name="s")

    @functools.partial(
        pl.kernel,
        out_type=jax.ShapeDtypeStruct((n_chunks * C_pad, D), jnp.float32),
        mesh=mesh,
        scratch_types=[
            pltpu.VMEM((KBLK,), jnp.int32),
            pltpu.VMEM((KBLK,), jnp.int32),
            pltpu.VMEM((CFLAT,), jnp.int32),
            pltpu.VMEM((CFLAT,), jnp.int32),
            pltpu.VMEM((NGMAX, G), jnp.int32),
            pltpu.VMEM((G, D), jnp.float32),
            pltpu.VMEM((G, D), jnp.float32),
            pltpu.VMEM_SHARED((C_pad, D), jnp.float32),
            pltpu.SemaphoreType.DMA,
            pltpu.SemaphoreType.DMA,
        ],
        compiler_params=pltpu.CompilerParams(use_tc_tiling_on_sc=False,
                                             needs_layout_passes=False),
    )
    def k(h_hbm, src_hbm, dst_hbm, agg_hbm,
          dblk, sblk, csrc, cdst, cdst2d, rows, rows2, chunk_sh, semA, semB):
        cid = lax.axis_index("c")
        sid = lax.axis_index("s")
        ebase = sid * EPT
        dump_row = C + sid
        pad_src = lax.iota(jnp.int32, 16) + sid * 16

        for p in range(n_chunks // NC):
            ch = cid + NC * p
            lo = ch * C
            row0 = sid * RPT

            @pl.loop(0, G)
            def _(r):
                for j in range(DV):
                    rows[r, pl.ds(j * 16, 16)] = jnp.zeros((16,), jnp.float32)

            @pl.loop(0, NOF)
            def _(z):
                pltpu.sync_copy(rows, chunk_sh.at[pl.ds(row0 + z * G, G)])
            if NOR:
                pltpu.sync_copy(rows.at[pl.ds(0, NOR)],
                                chunk_sh.at[pl.ds(row0 + NOF * G, NOR)])
            plsc.subcore_barrier()

            def block(b, carry):
                e0 = ebase + b * KBLK
                pltpu.sync_copy(src_hbm.at[pl.ds(e0, KBLK)], sblk)
                pltpu.sync_copy(dst_hbm.at[pl.ds(e0, KBLK)], dblk)

                def comp(i, m):
                    dv = dblk[pl.ds(i * 16, 16)]
                    sv = sblk[pl.ds(i * 16, 16)]
                    msk = (dv >= lo) & (dv < lo + C)
                    plsc.store_compressed(csrc.at[pl.ds(m, 16)], sv, mask=msk)
                    plsc.store_compressed(cdst.at[pl.ds(m, 16)], dv - lo, mask=msk)
                    return m + jnp.sum(msk.astype(jnp.int32))

                m = lax.fori_loop(0, KBLK // 16, comp, jnp.int32(0))
                ng = (m + (G - 1)) >> GSH
                npad = ((ng << GSH) - m + 15) >> 4
                pad_dst = jnp.full((16,), 0, jnp.int32) + dump_row

                def padb(i, _):
                    csrc[pl.ds(m + i * 16, 16)] = pad_src
                    cdst[pl.ds(m + i * 16, 16)] = pad_dst
                    return 0

                lax.fori_loop(0, npad, padb, 0)

                def cpb(i, _):
                    g = i >> (GSH - 4)
                    j = i & (G // 16 - 1)
                    cdst2d[g, pl.ds(j * 16, 16)] = cdst[pl.ds(i * 16, 16)]
                    return 0

                lax.fori_loop(0, ng << (GSH - 4), cpb, 0)

                def startA(g):
                    pltpu.make_async_copy(
                        h_hbm.at[csrc.at[pl.ds(g * G, G)]], rows, semA).start()

                def startB(g):
                    pltpu.make_async_copy(
                        h_hbm.at[csrc.at[pl.ds(g * G, G)]], rows2, semB).start()

                @pl.when(ng > 0)
                def _():
                    startA(0)

                @pl.when(ng > 1)
                def _():
                    startB(1)

                def pair(g2, _):
                    g = g2 * 2
                    pltpu.make_async_copy(
                        h_hbm.at[csrc.at[pl.ds(g * G, G)]], rows, semA).wait()
                    pltpu.sync_copy(rows, chunk_sh.at[cdst2d.at[g]], add=True)

                    @pl.when(g + 2 < ng)
                    def _():
                        startA(g + 2)

                    @pl.when(g + 1 < ng)
                    def _():
                        pltpu.make_async_copy(
                            h_hbm.at[csrc.at[pl.ds((g + 1) * G, G)]],
                            rows2, semB).wait()
                        pltpu.sync_copy(
                            rows2, chunk_sh.at[cdst2d.at[g + 1]], add=True)

                        @pl.when(g + 3 < ng)
                        def _():
                            startB(g + 3)

                    return 0

                lax.fori_loop(0, (ng + 1) >> 1, pair, 0)
                return carry

            lax.fori_loop(0, EPT // KBLK, block, 0)
            plsc.subcore_barrier()

            @pl.loop(0, NOF)
            def _(z):
                pltpu.sync_copy(chunk_sh.at[pl.ds(row0 + z * G, G)], rows)
                pltpu.sync_copy(
                    rows, agg_hbm.at[pl.ds(ch * C_pad + row0 + z * G, G)])
            if NOR:
                pltpu.sync_copy(chunk_sh.at[pl.ds(row0 + NOF * G, NOR)],
                                rows.at[pl.ds(0, NOR)])
                pltpu.sync_copy(
                    rows.at[pl.ds(0, NOR)],
                    agg_hbm.at[pl.ds(ch * C_pad + row0 + NOF * G, NOR)])
            plsc.subcore_barrier()

    return k


def _sc_agg(hprime, src, dst, D, C, n_chunks):
    PAD = 16 + ((16 - C % 16) % 16)
    C_pad = C + PAD
    out = _make_sc_agg(D, C, n_chunks)(hprime, src, dst)
    return out.reshape(n_chunks, C_pad, D)[:, :C, :].reshape(n_chunks * C, D)


RB = 2000


def _tc_mm(x, W, dout):
    din = x.shape[1]

    def body(x_ref, w_ref, o_ref):
        o_ref[...] = jnp.dot(x_ref[...], w_ref[...],
                             preferred_element_type=jnp.float32)

    return pl.pallas_call(
        body,
        grid=(NN // RB,),
        in_specs=[pl.BlockSpec((RB, din), lambda i: (i, 0)),
                  pl.BlockSpec((din, dout), lambda i: (0, 0))],
        out_specs=pl.BlockSpec((RB, dout), lambda i: (i, 0)),
        out_shape=jax.ShapeDtypeStruct((NN, dout), jnp.float32),
    )(x, W)


def _tc_scale(h, dinv2d):
    D = h.shape[1]

    def body(h_ref, dv_ref, o_ref):
        o_ref[...] = h_ref[...] * dv_ref[...]

    return pl.pallas_call(
        body,
        grid=(NN // RB,),
        in_specs=[pl.BlockSpec((RB, D), lambda i: (i, 0)),
                  pl.BlockSpec((RB, 1), lambda i: (i, 0))],
        out_specs=pl.BlockSpec((RB, D), lambda i: (i, 0)),
        out_shape=jax.ShapeDtypeStruct((NN, D), jnp.float32),
    )(h, dinv2d)


def _tc_junction(agg, hp, dinv2d, b, Wn):
    D = agg.shape[1]
    Dn = Wn.shape[1]

    def body(a_ref, h_ref, dv_ref, b_ref, w_ref, o_ref):
        dv = dv_ref[...]
        t = jax.nn.relu(dv * (a_ref[...] + h_ref[...]) + b_ref[...])
        o_ref[...] = jnp.dot(t, w_ref[...],
                             preferred_element_type=jnp.float32) * dv

    return pl.pallas_call(
        body,
        grid=(NN // RB,),
        in_specs=[pl.BlockSpec((RB, D), lambda i: (i, 0)),
                  pl.BlockSpec((RB, D), lambda i: (i, 0)),
                  pl.BlockSpec((RB, 1), lambda i: (i, 0)),
                  pl.BlockSpec((1, D), lambda i: (0, 0)),
                  pl.BlockSpec((D, Dn), lambda i: (0, 0))],
        out_specs=pl.BlockSpec((RB, Dn), lambda i: (i, 0)),
        out_shape=jax.ShapeDtypeStruct((NN, Dn), jnp.float32),
    )(agg, hp, dinv2d, b.reshape(1, D), Wn)


def _tc_post(agg, hp, dinv2d, b):
    D = agg.shape[1]

    def body(a_ref, h_ref, dv_ref, b_ref, o_ref):
        o_ref[...] = dv_ref[...] * (a_ref[...] + h_ref[...]) + b_ref[...]

    return pl.pallas_call(
        body,
        grid=(NN // RB,),
        in_specs=[pl.BlockSpec((RB, D), lambda i: (i, 0)),
                  pl.BlockSpec((RB, D), lambda i: (i, 0)),
                  pl.BlockSpec((RB, 1), lambda i: (i, 0)),
                  pl.BlockSpec((1, D), lambda i: (0, 0))],
        out_specs=pl.BlockSpec((RB, D), lambda i: (i, 0)),
        out_shape=jax.ShapeDtypeStruct((NN, D), jnp.float32),
    )(agg, hp, dinv2d, b.reshape(1, D))


def kernel(x, edge_index, W1, b1, W2, b2, W3, b3):
    src = edge_index[0]
    dst = edge_index[1]

    degp = _sc_degree(dst)
    deg = degp[:NN] + degp[DEG_PAD:DEG_PAD + NN] + 1.0
    dinv2d = lax.rsqrt(deg).reshape(NN, 1)

    h1p = _tc_scale(_tc_mm(x, W1, 16), dinv2d)
    agg1 = _sc_agg(h1p, src, dst, 16, 50000, 2)
    h2p = _tc_junction(agg1, h1p, dinv2d, b1, W2)
    agg2 = _sc_agg(h2p, src, dst, 32, 50000, 2)
    h3p = _tc_junction(agg2, h2p, dinv2d, b2, W3)
    agg3 = _sc_agg(h3p, src, dst, 64, 25000, 4)
    return _tc_post(agg3, h3p, dinv2d, b3)


# speedup vs baseline: 34.5927x; 1.3931x over previous
import functools

import jax
import jax.numpy as jnp
from jax import lax
from jax.experimental import pallas as pl
from jax.experimental.pallas import tpu as pltpu
from jax.experimental.pallas import tpu_sc as plsc

NN = 100000
EE = 1600000
NC = 2
NS = 16


DEG_PER_TILE = 6256
DEG_PAD = NS * DEG_PER_TILE
DEG_EPT = EE // (NC * NS)
DEG_BLK = 10000


def _sc_degree(ei):
    mesh = plsc.VectorSubcoreMesh(core_axis_name="c", subcore_axis_name="s")

    @functools.partial(
        pl.kernel,
        out_type=jax.ShapeDtypeStruct((NC * DEG_PAD,), jnp.float32),
        mesh=mesh,
        scratch_types=[
            pltpu.VMEM((DEG_BLK,), jnp.int32),
            pltpu.VMEM((DEG_BLK,), jnp.float32),
            pltpu.VMEM((DEG_PER_TILE,), jnp.float32),
            pltpu.VMEM_SHARED((DEG_PAD,), jnp.float32),
        ],
    )
    def k(ei_hbm, out_hbm, dblk, ones, zbuf, deg_sh):
        cid = lax.axis_index("c")
        sid = lax.axis_index("s")

        @pl.loop(0, DEG_BLK, step=16)
        def _(i):
            ones[pl.ds(i, 16)] = jnp.full((16,), 1.0, jnp.float32)

        @pl.loop(0, DEG_PER_TILE, step=16)
        def _(i):
            zbuf[pl.ds(i, 16)] = jnp.zeros((16,), jnp.float32)

        pltpu.sync_copy(zbuf, deg_sh.at[pl.ds(sid * DEG_PER_TILE, DEG_PER_TILE)])
        plsc.subcore_barrier()

        base = EE + (cid * NS + sid) * DEG_EPT

        @pl.loop(0, DEG_EPT, step=DEG_BLK)
        def _(e0):
            pltpu.sync_copy(ei_hbm.at[pl.ds(base + e0, DEG_BLK)], dblk)
            pltpu.sync_copy(ones, deg_sh.at[dblk], add=True)

        plsc.subcore_barrier()
        pltpu.sync_copy(deg_sh.at[pl.ds(sid * DEG_PER_TILE, DEG_PER_TILE)], zbuf)
        pltpu.sync_copy(
            zbuf,
            out_hbm.at[pl.ds(cid * DEG_PAD + sid * DEG_PER_TILE, DEG_PER_TILE)],
        )

    return k(ei)


EPT = EE // NS
KBLK = 2000
NB = EPT // KBLK


def _make_sc_agg(D, C, n_chunks):
    PAD = 16 + ((16 - C % 16) % 16)
    C_pad = C + PAD
    RPT = C_pad // NS
    G = 128 if D == 64 else 256
    GSH = G.bit_length() - 1
    NGMAX = (KBLK + G - 1) // G
    CFLAT = NGMAX * G + 16
    NZF, NZR = RPT // G, RPT % G
    NSTF, REM = C // G, C % G
    NSTT = (NSTF + NS - 1) // NS
    DV = D // 16
    mesh = plsc.VectorSubcoreMesh(core_axis_name="c", subcore_axis_name="s")

    @functools.partial(
        pl.kernel,
        out_type=jax.ShapeDtypeStruct((NN, D), jnp.float32),
        mesh=mesh,
        scratch_types=[
            pltpu.VMEM((KBLK,), jnp.int32),
            pltpu.VMEM((KBLK,), jnp.int32),
            pltpu.VMEM((KBLK,), jnp.int32),
            pltpu.VMEM((KBLK,), jnp.int32),
            pltpu.VMEM((CFLAT,), jnp.int32),
            pltpu.VMEM((CFLAT,), jnp.int32),
            pltpu.VMEM((G, D), jnp.float32),
            pltpu.VMEM((G, D), jnp.float32),
            pltpu.VMEM_SHARED((C_pad, D), jnp.float32),
            pltpu.SemaphoreType.DMA,
            pltpu.SemaphoreType.DMA,
            pltpu.SemaphoreType.DMA,
            pltpu.SemaphoreType.DMA,
        ],
        compiler_params=pltpu.CompilerParams(use_tc_tiling_on_sc=False,
                                             needs_layout_passes=False),
    )
    def k(h_hbm, ei_hbm, agg_hbm,
          dblkA, sblkA, dblkB, sblkB, csrc, cdst, rows, rows2, chunk_sh,
          semA, semB, semIA, semIB):
        cid = lax.axis_index("c")
        sid = lax.axis_index("s")
        ebase = sid * EPT
        dump_row = C + sid
        pad_src = lax.iota(jnp.int32, 16) + sid * 16

        def start_idx(e0, db, sb, semi):
            pltpu.make_async_copy(
                ei_hbm.at[pl.ds(EE + e0, KBLK)], db, semi).start()
            pltpu.make_async_copy(
                ei_hbm.at[pl.ds(e0, KBLK)], sb, semi).start()

        def wait_idx(db, sb, semi):
            pltpu.make_async_copy(
                ei_hbm.at[pl.ds(0, KBLK)], db, semi).wait()
            pltpu.make_async_copy(
                ei_hbm.at[pl.ds(0, KBLK)], sb, semi).wait()

        for p in range(n_chunks // NC):
            ch = cid + NC * p
            lo = ch * C
            row0 = sid * RPT

            @pl.loop(0, G)
            def _(r):
                for j in range(DV):
                    rows[r, pl.ds(j * 16, 16)] = jnp.zeros((16,), jnp.float32)

            @pl.loop(0, NZF)
            def _(z):
                pltpu.sync_copy(rows, chunk_sh.at[pl.ds(row0 + z * G, G)])
            if NZR:
                pltpu.sync_copy(rows.at[pl.ds(0, NZR)],
                                chunk_sh.at[pl.ds(row0 + NZF * G, NZR)])
            plsc.subcore_barrier()

            def process(db, sb):
                def comp(i, m):
                    dv = db[pl.ds(i * 16, 16)]
                    sv = sb[pl.ds(i * 16, 16)]
                    msk = (dv >= lo) & (dv < lo + C)
                    plsc.store_compressed(csrc.at[pl.ds(m, 16)], sv, mask=msk)
                    plsc.store_compressed(cdst.at[pl.ds(m, 16)], dv - lo,
                                          mask=msk)
                    return m + jnp.sum(msk.astype(jnp.int32))

                m = lax.fori_loop(0, KBLK // 16, comp, jnp.int32(0))
                ng = (m + (G - 1)) >> GSH
                npad = ((ng << GSH) - m + 15) >> 4
                pad_dst = jnp.full((16,), 0, jnp.int32) + dump_row

                def padb(i, _):
                    csrc[pl.ds(m + i * 16, 16)] = pad_src
                    cdst[pl.ds(m + i * 16, 16)] = pad_dst
                    return 0

                lax.fori_loop(0, npad, padb, 0)

                def startA(g):
                    pltpu.make_async_copy(
                        h_hbm.at[csrc.at[pl.ds(g * G, G)]], rows, semA).start()

                def startB(g):
                    pltpu.make_async_copy(
                        h_hbm.at[csrc.at[pl.ds(g * G, G)]], rows2, semB).start()

                @pl.when(ng > 0)
                def _():
                    startA(0)

                @pl.when(ng > 1)
                def _():
                    startB(1)

                def gpair(g2, _):
                    g = g2 * 2
                    pltpu.make_async_copy(
                        h_hbm.at[csrc.at[pl.ds(g * G, G)]], rows, semA).wait()
                    pltpu.sync_copy(
                        rows, chunk_sh.at[cdst.at[pl.ds(g * G, G)]], add=True)

                    @pl.when(g + 2 < ng)
                    def _():
                        startA(g + 2)

                    @pl.when(g + 1 < ng)
                    def _():
                        pltpu.make_async_copy(
                            h_hbm.at[csrc.at[pl.ds((g + 1) * G, G)]],
                            rows2, semB).wait()
                        pltpu.sync_copy(
                            rows2, chunk_sh.at[cdst.at[pl.ds((g + 1) * G, G)]],
                            add=True)

                        @pl.when(g + 3 < ng)
                        def _():
                            startB(g + 3)

                    return 0

                lax.fori_loop(0, (ng + 1) >> 1, gpair, 0)

            start_idx(ebase, dblkA, sblkA, semIA)

            def bpair(i, _):
                b = i * 2
                wait_idx(dblkA, sblkA, semIA)
                start_idx(ebase + (b + 1) * KBLK, dblkB, sblkB, semIB)
                process(dblkA, sblkA)
                wait_idx(dblkB, sblkB, semIB)

                @pl.when(b + 2 < NB)
                def _():
                    start_idx(ebase + (b + 2) * KBLK, dblkA, sblkA, semIA)

                process(dblkB, sblkB)
                return 0

            lax.fori_loop(0, NB // 2, bpair, 0)
            plsc.subcore_barrier()

            def outs(s2, _):
                s = sid + s2 * NS

                @pl.when(s < NSTF)
                def _():
                    pltpu.sync_copy(chunk_sh.at[pl.ds(s * G, G)], rows)
                    pltpu.sync_copy(rows, agg_hbm.at[pl.ds(lo + s * G, G)])
                return 0

            lax.fori_loop(0, NSTT, outs, 0)
            if REM:
                @pl.when(sid == NSTF % NS)
                def _():
                    pltpu.sync_copy(chunk_sh.at[pl.ds(NSTF * G, REM)],
                                    rows2.at[pl.ds(0, REM)])
                    pltpu.sync_copy(rows2.at[pl.ds(0, REM)],
                                    agg_hbm.at[pl.ds(lo + NSTF * G, REM)])
            plsc.subcore_barrier()

    return k


def _sc_agg(hprime, ei, D, C, n_chunks):
    return _make_sc_agg(D, C, n_chunks)(hprime, ei)


FB = 2048


def _tc_mm1(xp, Wb1, dv16):
    M = NN * 16 // 128
    ng = (M + FB - 1) // FB

    def body(x_ref, w_ref, dv_ref, o_ref):
        r = jnp.dot(x_ref[...], w_ref[...], preferred_element_type=jnp.float32)
        o_ref[...] = r * dv_ref[...]

    return pl.pallas_call(
        body,
        grid=(ng,),
        in_specs=[pl.BlockSpec((FB, 88), lambda i: (i, 0)),
                  pl.BlockSpec((88, 128), lambda i: (0, 0)),
                  pl.BlockSpec((FB, 128), lambda i: (i, 0))],
        out_specs=pl.BlockSpec((FB, 128), lambda i: (i, 0)),
        out_shape=jax.ShapeDtypeStruct((M, 128), jnp.float32),
    )(xp, Wb1, dv16)


def _tc_junction(af, hf, dvi, dvo, be, Wb):
    M = af.shape[0]
    ng = (M + FB - 1) // FB

    def body(a_ref, h_ref, dvi_ref, dvo_ref, b_ref, w_ref, o_ref):
        t = jax.nn.relu(dvi_ref[...] * (a_ref[...] + h_ref[...]) + b_ref[...])
        u = jnp.dot(t, w_ref[...], preferred_element_type=jnp.float32)
        o_ref[:, 0, :] = u[:, :128] * dvo_ref[:, 0, :]
        o_ref[:, 1, :] = u[:, 128:] * dvo_ref[:, 1, :]

    return pl.pallas_call(
        body,
        grid=(ng,),
        in_specs=[pl.BlockSpec((FB, 128), lambda i: (i, 0)),
                  pl.BlockSpec((FB, 128), lambda i: (i, 0)),
                  pl.BlockSpec((FB, 128), lambda i: (i, 0)),
                  pl.BlockSpec((FB, 2, 128), lambda i: (i, 0, 0)),
                  pl.BlockSpec((1, 128), lambda i: (0, 0)),
                  pl.BlockSpec((128, 256), lambda i: (0, 0))],
        out_specs=pl.BlockSpec((FB, 2, 128), lambda i: (i, 0, 0)),
        out_shape=jax.ShapeDtypeStruct((M, 2, 128), jnp.float32),
    )(af, hf, dvi, dvo, be, Wb)


def _tc_post(af, hf, dvo, be):
    M = af.shape[0]
    ng = (M + FB - 1) // FB

    def body(a_ref, h_ref, dv_ref, b_ref, o_ref):
        o_ref[...] = dv_ref[...] * (a_ref[...] + h_ref[...]) + b_ref[...]

    return pl.pallas_call(
        body,
        grid=(ng,),
        in_specs=[pl.BlockSpec((FB, 128), lambda i: (i, 0)),
                  pl.BlockSpec((FB, 128), lambda i: (i, 0)),
                  pl.BlockSpec((FB, 128), lambda i: (i, 0)),
                  pl.BlockSpec((1, 128), lambda i: (0, 0))],
        out_specs=pl.BlockSpec((FB, 128), lambda i: (i, 0)),
        out_shape=jax.ShapeDtypeStruct((M, 128), jnp.float32),
    )(af, hf, dvo, be)


def _block_diag(W, kk):
    D, Dn = W.shape
    out = jnp.zeros((kk * D, kk * Dn), W.dtype)
    for i in range(kk):
        out = out.at[i * D:(i + 1) * D, i * Dn:(i + 1) * Dn].set(W)
    return out


def kernel(x, edge_index, W1, b1, W2, b2, W3, b3):
    ei = edge_index.reshape(2 * EE)

    degp = _sc_degree(ei)
    deg = degp[:NN] + degp[DEG_PAD:DEG_PAD + NN] + 1.0
    dinv = lax.rsqrt(deg)
    dv16 = jnp.repeat(dinv, 16).reshape(-1, 128)
    dv32 = jnp.repeat(dinv, 32).reshape(-1, 2, 128)
    dv64 = jnp.repeat(dinv, 64).reshape(-1, 2, 128)
    b1e = jnp.tile(b1, 8).reshape(1, 128)
    b2e = jnp.tile(b2, 4).reshape(1, 128)
    b3e = jnp.tile(b3, 2).reshape(1, 128)
    xp = x.reshape(-1, 88)
    Wb1 = _block_diag(W1, 8)
    Wb2 = _block_diag(W2, 8)
    Wb3 = _block_diag(W3, 4)

    h1f = _tc_mm1(xp, Wb1, dv16)
    a1f = _sc_agg(h1f.reshape(NN, 16), ei, 16, 50000, 2).reshape(-1, 128)
    h2f = _tc_junction(a1f, h1f, dv16, dv32, b1e, Wb2).reshape(-1, 128)
    a2f = _sc_agg(h2f.reshape(NN, 32), ei, 32, 50000, 2).reshape(-1, 128)
    h3f = _tc_junction(a2f, h2f, dv32.reshape(-1, 128), dv64, b2e,
                       Wb3).reshape(-1, 128)
    a3f = _sc_agg(h3f.reshape(NN, 64), ei, 64, 25000, 4).reshape(-1, 128)
    return _tc_post(a3f, h3f, dv64.reshape(-1, 128),
                    b3e).reshape(NN, 64)


# speedup vs baseline: 37.9610x; 1.0974x over previous
import functools

import jax
import jax.numpy as jnp
from jax import lax
from jax.experimental import pallas as pl
from jax.experimental.pallas import tpu as pltpu
from jax.experimental.pallas import tpu_sc as plsc

NN = 100000
EE = 1600000
NC = 2
NS = 16


DEG_PER_TILE = 6256
DEG_PAD = NS * DEG_PER_TILE
DEG_EPT = EE // (NC * NS)
DEG_BLK = 10000


def _sc_degree(ei):
    mesh = plsc.VectorSubcoreMesh(core_axis_name="c", subcore_axis_name="s")

    @functools.partial(
        pl.kernel,
        out_type=jax.ShapeDtypeStruct((NC * DEG_PAD,), jnp.float32),
        mesh=mesh,
        scratch_types=[
            pltpu.VMEM((DEG_BLK,), jnp.int32),
            pltpu.VMEM((DEG_BLK,), jnp.float32),
            pltpu.VMEM((DEG_PER_TILE,), jnp.float32),
            pltpu.VMEM_SHARED((DEG_PAD,), jnp.float32),
        ],
    )
    def k(ei_hbm, out_hbm, dblk, ones, zbuf, deg_sh):
        cid = lax.axis_index("c")
        sid = lax.axis_index("s")

        @pl.loop(0, DEG_BLK, step=16)
        def _(i):
            ones[pl.ds(i, 16)] = jnp.full((16,), 1.0, jnp.float32)

        @pl.loop(0, DEG_PER_TILE, step=16)
        def _(i):
            zbuf[pl.ds(i, 16)] = jnp.zeros((16,), jnp.float32)

        pltpu.sync_copy(zbuf, deg_sh.at[pl.ds(sid * DEG_PER_TILE, DEG_PER_TILE)])
        plsc.subcore_barrier()

        base = EE + (cid * NS + sid) * DEG_EPT

        @pl.loop(0, DEG_EPT, step=DEG_BLK)
        def _(e0):
            pltpu.sync_copy(ei_hbm.at[pl.ds(base + e0, DEG_BLK)], dblk)
            pltpu.sync_copy(ones, deg_sh.at[dblk], add=True)

        plsc.subcore_barrier()
        pltpu.sync_copy(deg_sh.at[pl.ds(sid * DEG_PER_TILE, DEG_PER_TILE)], zbuf)
        pltpu.sync_copy(
            zbuf,
            out_hbm.at[pl.ds(cid * DEG_PAD + sid * DEG_PER_TILE, DEG_PER_TILE)],
        )

    return k(ei)


EPT = EE // NS
KBLK = 2000
NB = EPT // KBLK


def _make_sc_agg(D, C, n_chunks):
    PAD = 16 + ((16 - C % 16) % 16)
    C_pad = C + PAD
    RPT = C_pad // NS
    G = 128 if D == 64 else 256
    GSH = G.bit_length() - 1
    NGMAX = (KBLK + G - 1) // G
    CFLAT = NGMAX * G + 16
    NZF, NZR = RPT // G, RPT % G
    NSTF, REM = C // G, C % G
    NSTT = (NSTF + NS - 1) // NS
    DV = D // 16
    mesh = plsc.VectorSubcoreMesh(core_axis_name="c", subcore_axis_name="s")

    @functools.partial(
        pl.kernel,
        out_type=jax.ShapeDtypeStruct((NN, D), jnp.float32),
        mesh=mesh,
        scratch_types=[
            pltpu.VMEM((KBLK,), jnp.int32),
            pltpu.VMEM((KBLK,), jnp.int32),
            pltpu.VMEM((KBLK,), jnp.int32),
            pltpu.VMEM((KBLK,), jnp.int32),
            pltpu.VMEM((CFLAT,), jnp.int32),
            pltpu.VMEM((CFLAT,), jnp.int32),
            pltpu.VMEM((G, D), jnp.float32),
            pltpu.VMEM((G, D), jnp.float32),
            pltpu.VMEM_SHARED((C_pad, D), jnp.float32),
            pltpu.SemaphoreType.DMA,
            pltpu.SemaphoreType.DMA,
            pltpu.SemaphoreType.DMA,
            pltpu.SemaphoreType.DMA,
        ],
        compiler_params=pltpu.CompilerParams(use_tc_tiling_on_sc=False,
                                             needs_layout_passes=False),
    )
    def k(h_hbm, ei_hbm, agg_hbm,
          dblkA, sblkA, dblkB, sblkB, csrc, cdst,
          rows, rows2, chunk_sh,
          gs0, gs1, semIA, semIB):
        BUF = (rows, rows2)
        GS = (gs0, gs1)
        cid = lax.axis_index("c")
        sid = lax.axis_index("s")
        ebase = sid * EPT
        dump_row = C + sid
        pad_src = lax.iota(jnp.int32, 16) + sid * 16

        def start_idx(e0, db, sb, semi):
            pltpu.make_async_copy(
                ei_hbm.at[pl.ds(EE + e0, KBLK)], db, semi).start()
            pltpu.make_async_copy(
                ei_hbm.at[pl.ds(e0, KBLK)], sb, semi).start()

        def wait_idx(db, sb, semi):
            pltpu.make_async_copy(
                ei_hbm.at[pl.ds(0, KBLK)], db, semi).wait()
            pltpu.make_async_copy(
                ei_hbm.at[pl.ds(0, KBLK)], sb, semi).wait()

        for p in range(n_chunks // NC):
            ch = cid + NC * p
            lo = ch * C
            row0 = sid * RPT

            @pl.loop(0, G)
            def _(r):
                for j in range(DV):
                    rows[r, pl.ds(j * 16, 16)] = jnp.zeros((16,), jnp.float32)

            @pl.loop(0, NZF)
            def _(z):
                pltpu.sync_copy(rows, chunk_sh.at[pl.ds(row0 + z * G, G)])
            if NZR:
                pltpu.sync_copy(rows.at[pl.ds(0, NZR)],
                                chunk_sh.at[pl.ds(row0 + NZF * G, NZR)])
            plsc.subcore_barrier()

            def process(db, sb):
                def comp(i, mc):
                    dv = db[pl.ds(i * 16, 16)]
                    sv = sb[pl.ds(i * 16, 16)]
                    msk = (dv >= lo) & (dv < lo + C)
                    plsc.store_compressed(csrc.at[pl.ds(mc, 16)], sv, mask=msk)
                    plsc.store_compressed(cdst.at[pl.ds(mc, 16)], dv - lo,
                                          mask=msk)
                    return mc + jnp.sum(msk.astype(jnp.int32))

                m = lax.fori_loop(0, KBLK // 16, comp, jnp.int32(0))
                ng = (m + (G - 1)) >> GSH
                npad = ((ng << GSH) - m + 15) >> 4
                pad_dst = jnp.full((16,), 0, jnp.int32) + dump_row

                def padb(i, _):
                    csrc[pl.ds(m + i * 16, 16)] = pad_src
                    cdst[pl.ds(m + i * 16, 16)] = pad_dst
                    return 0

                lax.fori_loop(0, npad, padb, 0)

                def g_start(g, kk):
                    pltpu.make_async_copy(
                        h_hbm.at[csrc.at[pl.ds(g * G, G)]],
                        BUF[kk], GS[kk]).start()

                def g_wait(kk):
                    pltpu.make_async_copy(
                        h_hbm.at[csrc.at[pl.ds(0, G)]],
                        BUF[kk], GS[kk]).wait()

                @pl.when(ng > 0)
                def _():
                    g_start(0, 0)

                @pl.when(ng > 1)
                def _():
                    g_start(1, 1)

                def gpair(g2, _):
                    g = g2 * 2
                    g_wait(0)
                    pltpu.sync_copy(
                        BUF[0], chunk_sh.at[cdst.at[pl.ds(g * G, G)]],
                        add=True)

                    @pl.when(g + 2 < ng)
                    def _():
                        g_start(g + 2, 0)

                    @pl.when(g + 1 < ng)
                    def _():
                        g_wait(1)
                        pltpu.sync_copy(
                            BUF[1],
                            chunk_sh.at[cdst.at[pl.ds((g + 1) * G, G)]],
                            add=True)

                        @pl.when(g + 3 < ng)
                        def _():
                            g_start(g + 3, 1)

                    return 0

                lax.fori_loop(0, (ng + 1) >> 1, gpair, 0)

            start_idx(ebase, dblkA, sblkA, semIA)

            def bpair(i, _):
                b = i * 2
                wait_idx(dblkA, sblkA, semIA)
                start_idx(ebase + (b + 1) * KBLK, dblkB, sblkB, semIB)
                process(dblkA, sblkA)
                wait_idx(dblkB, sblkB, semIB)

                @pl.when(b + 2 < NB)
                def _():
                    start_idx(ebase + (b + 2) * KBLK, dblkA, sblkA, semIA)

                process(dblkB, sblkB)
                return 0

            lax.fori_loop(0, NB // 2, bpair, 0)
            plsc.subcore_barrier()

            def outs(s2, _):
                s = sid + s2 * NS

                @pl.when(s < NSTF)
                def _():
                    pltpu.sync_copy(chunk_sh.at[pl.ds(s * G, G)], rows)
                    pltpu.sync_copy(rows, agg_hbm.at[pl.ds(lo + s * G, G)])
                return 0

            lax.fori_loop(0, NSTT, outs, 0)
            if REM:
                @pl.when(sid == NSTF % NS)
                def _():
                    pltpu.sync_copy(chunk_sh.at[pl.ds(NSTF * G, REM)],
                                    rows2.at[pl.ds(0, REM)])
                    pltpu.sync_copy(rows2.at[pl.ds(0, REM)],
                                    agg_hbm.at[pl.ds(lo + NSTF * G, REM)])
            plsc.subcore_barrier()

    return k


def _sc_agg(hprime, ei, D, C, n_chunks):
    return _make_sc_agg(D, C, n_chunks)(hprime, ei)


FB = 2048


def _tc_mm1(xp, Wb1, d8, R16):
    M = NN * 16 // 128
    ng = (M + FB - 1) // FB

    def body(x_ref, w_ref, d_ref, r_ref, o_ref):
        r = jnp.dot(x_ref[...], w_ref[...], preferred_element_type=jnp.float32)
        dv = jnp.dot(d_ref[...], r_ref[...], preferred_element_type=jnp.float32,
                     precision=lax.Precision.HIGHEST)
        o_ref[...] = r * dv

    return pl.pallas_call(
        body,
        grid=(ng,),
        in_specs=[pl.BlockSpec((FB, 88), lambda i: (i, 0)),
                  pl.BlockSpec((88, 128), lambda i: (0, 0)),
                  pl.BlockSpec((FB, 8), lambda i: (i, 0)),
                  pl.BlockSpec((8, 128), lambda i: (0, 0))],
        out_specs=pl.BlockSpec((FB, 128), lambda i: (i, 0)),
        out_shape=jax.ShapeDtypeStruct((M, 128), jnp.float32),
    )(xp, Wb1, d8, R16)


def _tc_junction(af, hf, din, be, Wb, Rin, Rout):
    M = af.shape[0]
    ng = (M + FB - 1) // FB
    ki = din.shape[1]
    ko = ki // 2

    def body(a_ref, h_ref, di_ref, ri_ref, ro_ref, b_ref, w_ref, o_ref):
        d = di_ref[...]
        dvi = jnp.dot(d, ri_ref[...], preferred_element_type=jnp.float32,
                      precision=lax.Precision.HIGHEST)
        t = jax.nn.relu(dvi * (a_ref[...] + h_ref[...]) + b_ref[...])
        u = jnp.dot(t, w_ref[...], preferred_element_type=jnp.float32)
        dvo_lo = jnp.dot(d[:, :ko], ro_ref[...],
                         preferred_element_type=jnp.float32,
                         precision=lax.Precision.HIGHEST)
        dvo_hi = jnp.dot(d[:, ko:], ro_ref[...],
                         preferred_element_type=jnp.float32,
                         precision=lax.Precision.HIGHEST)
        o_ref[:, 0, :] = u[:, :128] * dvo_lo
        o_ref[:, 1, :] = u[:, 128:] * dvo_hi

    return pl.pallas_call(
        body,
        grid=(ng,),
        in_specs=[pl.BlockSpec((FB, 128), lambda i: (i, 0)),
                  pl.BlockSpec((FB, 128), lambda i: (i, 0)),
                  pl.BlockSpec((FB, ki), lambda i: (i, 0)),
                  pl.BlockSpec((ki, 128), lambda i: (0, 0)),
                  pl.BlockSpec((ko, 128), lambda i: (0, 0)),
                  pl.BlockSpec((1, 128), lambda i: (0, 0)),
                  pl.BlockSpec((128, 256), lambda i: (0, 0))],
        out_specs=pl.BlockSpec((FB, 2, 128), lambda i: (i, 0, 0)),
        out_shape=jax.ShapeDtypeStruct((M, 2, 128), jnp.float32),
    )(af, hf, din, Rin, Rout, be, Wb)


def _tc_post(af, hf, d2, R64, be):
    M = af.shape[0]
    ng = (M + FB - 1) // FB

    def body(a_ref, h_ref, d_ref, r_ref, b_ref, o_ref):
        dv = jnp.dot(d_ref[...], r_ref[...], preferred_element_type=jnp.float32,
                     precision=lax.Precision.HIGHEST)
        o_ref[...] = dv * (a_ref[...] + h_ref[...]) + b_ref[...]

    return pl.pallas_call(
        body,
        grid=(ng,),
        in_specs=[pl.BlockSpec((FB, 128), lambda i: (i, 0)),
                  pl.BlockSpec((FB, 128), lambda i: (i, 0)),
                  pl.BlockSpec((FB, 2), lambda i: (i, 0)),
                  pl.BlockSpec((2, 128), lambda i: (0, 0)),
                  pl.BlockSpec((1, 128), lambda i: (0, 0))],
        out_specs=pl.BlockSpec((FB, 128), lambda i: (i, 0)),
        out_shape=jax.ShapeDtypeStruct((M, 128), jnp.float32),
    )(af, hf, d2, R64, be)


def _block_diag(W, kk):
    D, Dn = W.shape
    out = jnp.zeros((kk * D, kk * Dn), W.dtype)
    for i in range(kk):
        out = out.at[i * D:(i + 1) * D, i * Dn:(i + 1) * Dn].set(W)
    return out


def kernel(x, edge_index, W1, b1, W2, b2, W3, b3):
    ei = edge_index.reshape(2 * EE)

    degp = _sc_degree(ei)
    deg = degp[:NN] + degp[DEG_PAD:DEG_PAD + NN] + 1.0
    dinv = lax.rsqrt(deg)
    d8 = dinv.reshape(-1, 8)
    d4 = dinv.reshape(-1, 4)
    d2 = dinv.reshape(-1, 2)
    eye = jnp.eye(8, dtype=jnp.float32)
    R16 = jnp.repeat(eye, 16, axis=1)
    R32 = jnp.repeat(eye[:4, :4], 32, axis=1)
    R64 = jnp.repeat(eye[:2, :2], 64, axis=1)
    b1e = jnp.tile(b1, 8).reshape(1, 128)
    b2e = jnp.tile(b2, 4).reshape(1, 128)
    b3e = jnp.tile(b3, 2).reshape(1, 128)
    xp = x.reshape(-1, 88)
    Wb1 = _block_diag(W1, 8)
    Wb2 = _block_diag(W2, 8)
    Wb3 = _block_diag(W3, 4)

    h1f = _tc_mm1(xp, Wb1, d8, R16)
    a1f = _sc_agg(h1f.reshape(NN, 16), ei, 16, 50000, 2).reshape(-1, 128)
    h2f = _tc_junction(a1f, h1f, d8, b1e, Wb2, R16, R32).reshape(-1, 128)
    a2f = _sc_agg(h2f.reshape(NN, 32), ei, 32, 50000, 2).reshape(-1, 128)
    h3f = _tc_junction(a2f, h2f, d4, b2e, Wb3, R32, R64).reshape(-1, 128)
    a3f = _sc_agg(h3f.reshape(NN, 64), ei, 64, 25000, 4).reshape(-1, 128)
    return _tc_post(a3f, h3f, d2, R64, b3e).reshape(NN, 64)
